# baseline (speedup 1.0000x reference)
import functools

import jax
import jax.numpy as jnp
from jax import lax
from jax.experimental import pallas as pl
from jax.experimental.pallas import tpu as pltpu
from jax.experimental.pallas import tpu_sc as plsc

B, N, M, C1, C2 = 8, 4096, 1024, 64, 128
TN = 512
BN = B * N
NW = 32
PW = BN // NW
CP = 128
NCHUNK = PW // CP


def _three_nn_kernel(unknown_ref, known_ref, idx_ref, w_ref):
    b = pl.program_id(0)
    u = unknown_ref[0]
    k = known_ref[0]

    u2 = jnp.sum(u * u, axis=-1, keepdims=True)
    k2 = jnp.sum(k * k, axis=-1)[None, :]
    cross = jax.lax.dot_general(
        -2.0 * u, k, (((1,), (1,)), ((), ())),
        preferred_element_type=jnp.float32)
    d2 = (u2 + cross) + k2

    big = jnp.float32(jnp.inf)
    iota = jax.lax.broadcasted_iota(jnp.int32, (TN, M), 1)

    def amin(d):
        m = jnp.min(d, axis=-1, keepdims=True)
        eq = d == m
        i = jnp.min(jnp.where(eq, iota, M), axis=-1, keepdims=True)
        return m, i, jnp.where(eq, big, d)

    m0, i0, d2a = amin(d2)
    m1, i1, d2b = amin(d2a)
    m2, i2, _ = amin(d2b)

    r0 = 1.0 / (jnp.maximum(m0, 0.0) + 1e-8)
    r1 = 1.0 / (jnp.maximum(m1, 0.0) + 1e-8)
    r2 = 1.0 / (jnp.maximum(m2, 0.0) + 1e-8)
    inv_norm = 1.0 / (r0 + r1 + r2)

    base = b * M
    idx_ref[0, 0, 0, :] = (i0 + base)[:, 0]
    idx_ref[1, 0, 0, :] = (i1 + base)[:, 0]
    idx_ref[2, 0, 0, :] = (i2 + base)[:, 0]
    w_ref[0, 0, :, :] = jnp.broadcast_to(r0 * inv_norm, (TN, 16))
    w_ref[1, 0, :, :] = jnp.broadcast_to(r1 * inv_norm, (TN, 16))
    w_ref[2, 0, :, :] = jnp.broadcast_to(r2 * inv_norm, (TN, 16))


def _three_nn(unknown, known):
    grid = (B, N // TN)
    return pl.pallas_call(
        _three_nn_kernel,
        grid=grid,
        in_specs=[
            pl.BlockSpec((1, TN, 3), lambda b, i: (b, i, 0)),
            pl.BlockSpec((1, M, 3), lambda b, i: (b, 0, 0)),
        ],
        out_specs=[
            pl.BlockSpec((3, 1, 1, TN),
                         lambda b, i: (0, b * (N // TN) + i, 0, 0)),
            pl.BlockSpec((3, 1, TN, 16),
                         lambda b, i: (0, b * (N // TN) + i, 0, 0)),
        ],
        out_shape=[
            jax.ShapeDtypeStruct((3, BN // TN, 1, TN), jnp.int32),
            jax.ShapeDtypeStruct((3, BN // TN, TN, 16), jnp.float32),
        ],
    )(unknown, known)


@functools.cache
def _build_sc_interp():
    mesh = plsc.VectorSubcoreMesh(core_axis_name="c", subcore_axis_name="s")

    @functools.partial(
        pl.kernel,
        mesh=mesh,
        out_type=jax.ShapeDtypeStruct((BN, C1 + C2), jnp.float32),
        scratch_types=[
        pltpu.VMEM((CP,), jnp.int32),
        pltpu.VMEM((CP,), jnp.int32),
        pltpu.VMEM((CP,), jnp.int32),
        pltpu.VMEM((CP * 16,), jnp.float32),
        pltpu.VMEM((CP * 16,), jnp.float32),
        pltpu.VMEM((CP * 16,), jnp.float32),
        pltpu.VMEM((CP, C2), jnp.float32),
        pltpu.VMEM((CP, C2), jnp.float32),
        pltpu.VMEM((CP, C2), jnp.float32),
        pltpu.VMEM((CP, C2), jnp.float32),
            pltpu.VMEM((CP, C1), jnp.float32),
            pltpu.SemaphoreType.DMA,
        ],
    )
    def _sc_interp(table_hbm, idx_hbm, w_hbm, uf_hbm, out_hbm,
                   i0_v, i1_v, i2_v, w0_v, w1_v, w2_v,
                   g0_v, g1_v, g2_v, acc_v, uf_v, sem):
        wid = lax.axis_index("s") * 2 + lax.axis_index("c")
        wbase = wid * PW

        def chunk_body(t, carry):
            base = wbase + t * CP
            pltpu.sync_copy(idx_hbm.at[pl.ds(base, CP)], i0_v)
            pltpu.sync_copy(idx_hbm.at[pl.ds(BN + base, CP)], i1_v)
            pltpu.sync_copy(idx_hbm.at[pl.ds(2 * BN + base, CP)], i2_v)
            pltpu.sync_copy(w_hbm.at[pl.ds(base * 16, CP * 16)], w0_v)
            pltpu.sync_copy(w_hbm.at[pl.ds((BN + base) * 16, CP * 16)], w1_v)
            pltpu.sync_copy(w_hbm.at[pl.ds((2 * BN + base) * 16, CP * 16)], w2_v)
            pltpu.sync_copy(uf_hbm.at[pl.ds(base, CP), :], uf_v)
            a0 = pltpu.async_copy(table_hbm.at[i0_v], g0_v, sem)
            a1 = pltpu.async_copy(table_hbm.at[i1_v], g1_v, sem)
            a2 = pltpu.async_copy(table_hbm.at[i2_v], g2_v, sem)
            a0.wait()
            a1.wait()
            a2.wait()

            def point_body(p, carry2):
                w0 = w0_v[pl.ds(p * 16, 16)]
                w1 = w1_v[pl.ds(p * 16, 16)]
                w2 = w2_v[pl.ds(p * 16, 16)]
                for f in range(C2 // 16):
                    sl = pl.ds(f * 16, 16)
                    acc_v[p, sl] = (w0 * g0_v[p, sl] + w1 * g1_v[p, sl]
                                    + w2 * g2_v[p, sl])
                return carry2

            lax.fori_loop(0, CP, point_body, 0)
            pltpu.sync_copy(acc_v, out_hbm.at[pl.ds(base, CP), pl.ds(0, C2)])
            pltpu.sync_copy(uf_v, out_hbm.at[pl.ds(base, CP), pl.ds(C2, C1)])
            return carry

        lax.fori_loop(0, NCHUNK, chunk_body, 0)

    return _sc_interp


@jax.jit
def kernel(unknown, known, unknow_feats, known_feats):
    idxg, wts = _three_nn(unknown, known)
    table = known_feats.reshape(B * M, C2)
    idx_flat = idxg.reshape(3 * BN)
    w_flat = wts.reshape(3 * BN * 16)
    uf_flat = unknow_feats.reshape(BN, C1)
    out = _build_sc_interp()(table, idx_flat, w_flat, uf_flat)
    out = out.reshape(B, N, C1 + C2)
    return (out, out)


# speedup vs baseline: 1.2098x; 1.2098x over previous
import functools

import jax
import jax.numpy as jnp
from jax import lax
from jax.experimental import pallas as pl
from jax.experimental.pallas import tpu as pltpu
from jax.experimental.pallas import tpu_sc as plsc

B, N, M, C1, C2 = 8, 4096, 1024, 64, 128
TN = 512
BN = B * N
NW = 32
PW = BN // NW
CP = 128
NCHUNK = PW // CP

_BIG = 1e30


def _three_nn_kernel(unknown_ref, known_ref,
                     i0_ref, i1_ref, i2_ref, w0_ref, w1_ref, w2_ref):
    b = pl.program_id(0)
    u = unknown_ref[0]
    k = known_ref[0]

    ut = jnp.transpose(-2.0 * u)
    u2 = 0.25 * jnp.sum(ut * ut, axis=0, keepdims=True)
    k2 = jnp.sum(k * k, axis=-1, keepdims=True)
    cross = jax.lax.dot_general(
        k, ut, (((1,), (0,)), ((), ())),
        preferred_element_type=jnp.float32)
    d2 = (u2 + cross) + k2

    iota = jax.lax.broadcasted_iota(jnp.int32, (M, TN), 0)

    def pick(d):
        m = jnp.min(d, axis=0, keepdims=True)
        eq = d == m
        idx = jnp.min(jnp.where(eq, iota, M), axis=0, keepdims=True)
        return m, idx, jnp.where(eq, _BIG, d)

    m0, i0, d2a = pick(d2)
    m1, i1, d2b = pick(d2a)
    m2, i2, _ = pick(d2b)

    r0 = 1.0 / (jnp.maximum(m0, 0.0) + 1e-8)
    r1 = 1.0 / (jnp.maximum(m1, 0.0) + 1e-8)
    r2 = 1.0 / (jnp.maximum(m2, 0.0) + 1e-8)
    inv_norm = 1.0 / (r0 + r1 + r2)

    base = b * M
    i0_ref[:] = (i0 + base)[0, :]
    i1_ref[:] = (i1 + base)[0, :]
    i2_ref[:] = (i2 + base)[0, :]
    w0_ref[:] = (r0 * inv_norm)[0, :]
    w1_ref[:] = (r1 * inv_norm)[0, :]
    w2_ref[:] = (r2 * inv_norm)[0, :]


def _three_nn(unknown, known):
    grid = (B, N // TN)
    flat_spec = pl.BlockSpec((TN,), lambda b, i: (b * (N // TN) + i,))
    return pl.pallas_call(
        _three_nn_kernel,
        grid=grid,
        in_specs=[
            pl.BlockSpec((1, TN, 3), lambda b, i: (b, i, 0)),
            pl.BlockSpec((1, M, 3), lambda b, i: (b, 0, 0)),
        ],
        out_specs=[flat_spec] * 6,
        out_shape=[jax.ShapeDtypeStruct((BN,), jnp.int32)] * 3
        + [jax.ShapeDtypeStruct((BN,), jnp.float32)] * 3,
    )(unknown, known)


_GDN = lax.GatherDimensionNumbers(
    offset_dims=(), collapsed_slice_dims=(0,), start_index_map=(0,))


def _bcast(vec16, j):
    jv = jnp.full((16, 1), j, jnp.int32)
    return lax.gather(vec16, jv, _GDN, slice_sizes=(1,),
                      mode=lax.GatherScatterMode.PROMISE_IN_BOUNDS)


@functools.cache
def _build_sc_interp():
    mesh = plsc.VectorSubcoreMesh(core_axis_name="c", subcore_axis_name="s")

    @functools.partial(
        pl.kernel,
        mesh=mesh,
        out_type=jax.ShapeDtypeStruct((BN, C1 + C2), jnp.float32),
        scratch_types=[
            pltpu.VMEM((CP,), jnp.int32),
            pltpu.VMEM((CP,), jnp.int32),
            pltpu.VMEM((CP,), jnp.int32),
            pltpu.VMEM((CP,), jnp.float32),
            pltpu.VMEM((CP,), jnp.float32),
            pltpu.VMEM((CP,), jnp.float32),
            pltpu.VMEM((CP, C2), jnp.float32),
            pltpu.VMEM((CP, C2), jnp.float32),
            pltpu.VMEM((CP, C2), jnp.float32),
            pltpu.VMEM((CP, C2), jnp.float32),
            pltpu.VMEM((CP, C1), jnp.float32),
            pltpu.SemaphoreType.DMA,
        ],
    )
    def _sc_interp(table_hbm, i0_hbm, i1_hbm, i2_hbm, w0_hbm, w1_hbm, w2_hbm,
                   uf_hbm, out_hbm,
                   i0_v, i1_v, i2_v, w0_v, w1_v, w2_v,
                   g0_v, g1_v, g2_v, acc_v, uf_v, sem):
        wid = lax.axis_index("s") * 2 + lax.axis_index("c")
        wbase = wid * PW

        def chunk_body(t, carry):
            base = wbase + t * CP
            pltpu.sync_copy(i0_hbm.at[pl.ds(base, CP)], i0_v)
            pltpu.sync_copy(i1_hbm.at[pl.ds(base, CP)], i1_v)
            pltpu.sync_copy(i2_hbm.at[pl.ds(base, CP)], i2_v)
            pltpu.sync_copy(w0_hbm.at[pl.ds(base, CP)], w0_v)
            pltpu.sync_copy(w1_hbm.at[pl.ds(base, CP)], w1_v)
            pltpu.sync_copy(w2_hbm.at[pl.ds(base, CP)], w2_v)
            pltpu.sync_copy(uf_hbm.at[pl.ds(base, CP), :], uf_v)
            a0 = pltpu.async_copy(table_hbm.at[i0_v], g0_v, sem)
            a1 = pltpu.async_copy(table_hbm.at[i1_v], g1_v, sem)
            a2 = pltpu.async_copy(table_hbm.at[i2_v], g2_v, sem)
            a0.wait()
            a1.wait()
            a2.wait()

            def group_body(q, carry2):
                wq0 = w0_v[pl.ds(q * 16, 16)]
                wq1 = w1_v[pl.ds(q * 16, 16)]
                wq2 = w2_v[pl.ds(q * 16, 16)]
                for j in range(16):
                    p = q * 16 + j
                    w0 = _bcast(wq0, j)
                    w1 = _bcast(wq1, j)
                    w2 = _bcast(wq2, j)
                    for f in range(C2 // 16):
                        sl = pl.ds(f * 16, 16)
                        acc_v[p, sl] = (w0 * g0_v[p, sl] + w1 * g1_v[p, sl]
                                        + w2 * g2_v[p, sl])
                return carry2

            lax.fori_loop(0, CP // 16, group_body, 0)
            pltpu.sync_copy(acc_v, out_hbm.at[pl.ds(base, CP), pl.ds(0, C2)])
            pltpu.sync_copy(uf_v, out_hbm.at[pl.ds(base, CP), pl.ds(C2, C1)])
            return carry

        lax.fori_loop(0, NCHUNK, chunk_body, 0)

    return _sc_interp


@jax.jit
def kernel(unknown, known, unknow_feats, known_feats):
    i0, i1, i2, w0, w1, w2 = _three_nn(unknown, known)
    table = known_feats.reshape(B * M, C2)
    uf_flat = unknow_feats.reshape(BN, C1)
    out = _build_sc_interp()(table, i0, i1, i2, w0, w1, w2, uf_flat)
    out = out.reshape(B, N, C1 + C2)
    return (out, out)


# speedup vs baseline: 1.2581x; 1.0400x over previous
import functools

import jax
import jax.numpy as jnp
from jax import lax
from jax.experimental import pallas as pl
from jax.experimental.pallas import tpu as pltpu
from jax.experimental.pallas import tpu_sc as plsc

B, N, M, C1, C2 = 8, 4096, 1024, 64, 128
TN = 512
BN = B * N
NW = 32
PW = BN // NW
CP = 128
NCHUNK = PW // CP

_BIG = 1e30


def _three_nn_kernel(b0, unknown_ref, known_ref,
                     i0_ref, i1_ref, i2_ref, w0_ref, w1_ref, w2_ref):
    b = pl.program_id(0) + b0
    u = unknown_ref[0]
    k = known_ref[0]

    ut = jnp.transpose(-2.0 * u)
    u2 = 0.25 * jnp.sum(ut * ut, axis=0, keepdims=True)
    k2 = jnp.sum(k * k, axis=-1, keepdims=True)
    cross = jax.lax.dot_general(
        k, ut, (((1,), (0,)), ((), ())),
        preferred_element_type=jnp.float32)
    d2 = (u2 + cross) + k2

    iota = jax.lax.broadcasted_iota(jnp.int32, (M, TN), 0)

    def pick(d):
        m = jnp.min(d, axis=0, keepdims=True)
        eq = d == m
        idx = jnp.min(jnp.where(eq, iota, M), axis=0, keepdims=True)
        return m, idx, jnp.where(eq, _BIG, d)

    m0, i0, d2a = pick(d2)
    m1, i1, d2b = pick(d2a)
    m2, i2, _ = pick(d2b)

    r0 = 1.0 / (jnp.maximum(m0, 0.0) + 1e-8)
    r1 = 1.0 / (jnp.maximum(m1, 0.0) + 1e-8)
    r2 = 1.0 / (jnp.maximum(m2, 0.0) + 1e-8)
    inv_norm = 1.0 / (r0 + r1 + r2)

    base = b * M
    i0_ref[:] = (i0 + base)[0, :]
    i1_ref[:] = (i1 + base)[0, :]
    i2_ref[:] = (i2 + base)[0, :]
    w0_ref[:] = (r0 * inv_norm)[0, :]
    w1_ref[:] = (r1 * inv_norm)[0, :]
    w2_ref[:] = (r2 * inv_norm)[0, :]


def _three_nn(unknown, known, b0, nb):
    grid = (nb, N // TN)
    npts = nb * N
    flat_spec = pl.BlockSpec((TN,), lambda b, i: (b * (N // TN) + i,))
    return pl.pallas_call(
        functools.partial(_three_nn_kernel, b0),
        grid=grid,
        in_specs=[
            pl.BlockSpec((1, TN, 3), lambda b, i: (b, i, 0)),
            pl.BlockSpec((1, M, 3), lambda b, i: (b, 0, 0)),
        ],
        out_specs=[flat_spec] * 6,
        out_shape=[jax.ShapeDtypeStruct((npts,), jnp.int32)] * 3
        + [jax.ShapeDtypeStruct((npts,), jnp.float32)] * 3,
    )(unknown, known)


_GDN = lax.GatherDimensionNumbers(
    offset_dims=(), collapsed_slice_dims=(0,), start_index_map=(0,))


def _bcast(vec16, j):
    jv = jnp.full((16, 1), j, jnp.int32)
    return lax.gather(vec16, jv, _GDN, slice_sizes=(1,),
                      mode=lax.GatherScatterMode.PROMISE_IN_BOUNDS)


@functools.cache
def _build_sc_interp(npts):
    pw = npts // NW
    nchunk = pw // CP
    mesh = plsc.VectorSubcoreMesh(core_axis_name="c", subcore_axis_name="s")

    @functools.partial(
        pl.kernel,
        mesh=mesh,
        out_type=jax.ShapeDtypeStruct((npts, C1 + C2), jnp.float32),
        scratch_types=[
            pltpu.VMEM((CP,), jnp.int32),
            pltpu.VMEM((CP,), jnp.int32),
            pltpu.VMEM((CP,), jnp.int32),
            pltpu.VMEM((CP,), jnp.float32),
            pltpu.VMEM((CP,), jnp.float32),
            pltpu.VMEM((CP,), jnp.float32),
            pltpu.VMEM((CP, C2), jnp.float32),
            pltpu.VMEM((CP, C2), jnp.float32),
            pltpu.VMEM((CP, C2), jnp.float32),
            pltpu.VMEM((CP, C2), jnp.float32),
            pltpu.VMEM((CP, C1), jnp.float32),
            pltpu.SemaphoreType.DMA,
        ],
    )
    def _sc_interp(table_hbm, i0_hbm, i1_hbm, i2_hbm, w0_hbm, w1_hbm, w2_hbm,
                   uf_hbm, out_hbm,
                   i0_v, i1_v, i2_v, w0_v, w1_v, w2_v,
                   g0_v, g1_v, g2_v, acc_v, uf_v, sem):
        wid = lax.axis_index("s") * 2 + lax.axis_index("c")
        wbase = wid * pw

        def chunk_body(t, carry):
            base = wbase + t * CP
            pltpu.sync_copy(i0_hbm.at[pl.ds(base, CP)], i0_v)
            pltpu.sync_copy(i1_hbm.at[pl.ds(base, CP)], i1_v)
            pltpu.sync_copy(i2_hbm.at[pl.ds(base, CP)], i2_v)
            pltpu.sync_copy(w0_hbm.at[pl.ds(base, CP)], w0_v)
            pltpu.sync_copy(w1_hbm.at[pl.ds(base, CP)], w1_v)
            pltpu.sync_copy(w2_hbm.at[pl.ds(base, CP)], w2_v)
            pltpu.sync_copy(uf_hbm.at[pl.ds(base, CP), :], uf_v)
            a0 = pltpu.async_copy(table_hbm.at[i0_v], g0_v, sem)
            a1 = pltpu.async_copy(table_hbm.at[i1_v], g1_v, sem)
            a2 = pltpu.async_copy(table_hbm.at[i2_v], g2_v, sem)
            a0.wait()
            a1.wait()
            a2.wait()

            def group_body(q, carry2):
                wq0 = w0_v[pl.ds(q * 16, 16)]
                wq1 = w1_v[pl.ds(q * 16, 16)]
                wq2 = w2_v[pl.ds(q * 16, 16)]
                for j in range(16):
                    p = q * 16 + j
                    w0 = _bcast(wq0, j)
                    w1 = _bcast(wq1, j)
                    w2 = _bcast(wq2, j)
                    for f in range(C2 // 16):
                        sl = pl.ds(f * 16, 16)
                        acc_v[p, sl] = (w0 * g0_v[p, sl] + w1 * g1_v[p, sl]
                                        + w2 * g2_v[p, sl])
                return carry2

            lax.fori_loop(0, CP // 16, group_body, 0)
            pltpu.sync_copy(acc_v, out_hbm.at[pl.ds(base, CP), pl.ds(0, C2)])
            pltpu.sync_copy(uf_v, out_hbm.at[pl.ds(base, CP), pl.ds(C2, C1)])
            return carry

        lax.fori_loop(0, nchunk, chunk_body, 0)

    return _sc_interp


NSPLIT = 2
HB = B // NSPLIT
HP = HB * N


@jax.jit
def kernel(unknown, known, unknow_feats, known_feats):
    table = known_feats.reshape(B * M, C2)
    uf_flat = unknow_feats.reshape(BN, C1)
    sc = _build_sc_interp(HP)
    parts = []
    for h in range(NSPLIT):
        bs = slice(h * HB, (h + 1) * HB)
        i0, i1, i2, w0, w1, w2 = _three_nn(unknown[bs], known[bs], h * HB, HB)
        parts.append(
            sc(table, i0, i1, i2, w0, w1, w2,
               uf_flat[h * HP:(h + 1) * HP]))
    out = jnp.concatenate(parts, axis=0).reshape(B, N, C1 + C2)
    return (out, out)


# speedup vs baseline: 1.3122x; 1.0430x over previous
import functools

import jax
import jax.numpy as jnp
from jax import lax
from jax.experimental import pallas as pl
from jax.experimental.pallas import tpu as pltpu
from jax.experimental.pallas import tpu_sc as plsc

B, N, M, C1, C2 = 8, 4096, 1024, 64, 128
TN = 512
BN = B * N
NW = 32
PW = BN // NW
CP = 128
NCHUNK = PW // CP

_BIG = 1e30


def _three_nn_kernel(b0, unknown_ref, known_ref,
                     i0_ref, i1_ref, i2_ref, w0_ref, w1_ref, w2_ref):
    b = pl.program_id(0) + b0
    u = unknown_ref[0]
    k = known_ref[0]

    ut = jnp.transpose(-2.0 * u)
    u2 = 0.25 * jnp.sum(ut * ut, axis=0, keepdims=True)
    k2 = jnp.sum(k * k, axis=-1, keepdims=True)
    cross = jax.lax.dot_general(
        k, ut, (((1,), (0,)), ((), ())),
        preferred_element_type=jnp.float32)
    d2 = (u2 + cross) + k2

    iota = jax.lax.broadcasted_iota(jnp.int32, (M, TN), 0)

    def pick(d):
        m = jnp.min(d, axis=0, keepdims=True)
        eq = d == m
        idx = jnp.min(jnp.where(eq, iota, M), axis=0, keepdims=True)
        return m, idx, jnp.where(eq, _BIG, d)

    m0, i0, d2a = pick(d2)
    m1, i1, d2b = pick(d2a)
    m2, i2, _ = pick(d2b)

    r0 = 1.0 / (jnp.maximum(m0, 0.0) + 1e-8)
    r1 = 1.0 / (jnp.maximum(m1, 0.0) + 1e-8)
    r2 = 1.0 / (jnp.maximum(m2, 0.0) + 1e-8)
    inv_norm = 1.0 / (r0 + r1 + r2)

    base = b * M
    i0_ref[:] = (i0 + base)[0, :]
    i1_ref[:] = (i1 + base)[0, :]
    i2_ref[:] = (i2 + base)[0, :]
    w0_ref[:] = (r0 * inv_norm)[0, :]
    w1_ref[:] = (r1 * inv_norm)[0, :]
    w2_ref[:] = (r2 * inv_norm)[0, :]


def _three_nn(unknown, known, b0, nb):
    grid = (nb, N // TN)
    npts = nb * N
    flat_spec = pl.BlockSpec((TN,), lambda b, i: (b * (N // TN) + i,))
    return pl.pallas_call(
        functools.partial(_three_nn_kernel, b0),
        grid=grid,
        in_specs=[
            pl.BlockSpec((1, TN, 3), lambda b, i: (b, i, 0)),
            pl.BlockSpec((1, M, 3), lambda b, i: (b, 0, 0)),
        ],
        out_specs=[flat_spec] * 6,
        out_shape=[jax.ShapeDtypeStruct((npts,), jnp.int32)] * 3
        + [jax.ShapeDtypeStruct((npts,), jnp.float32)] * 3,
    )(unknown, known)


_GDN = lax.GatherDimensionNumbers(
    offset_dims=(), collapsed_slice_dims=(0,), start_index_map=(0,))


def _bcast(vec16, j):
    jv = jnp.full((16, 1), j, jnp.int32)
    return lax.gather(vec16, jv, _GDN, slice_sizes=(1,),
                      mode=lax.GatherScatterMode.PROMISE_IN_BOUNDS)


@functools.cache
def _build_sc_interp(npts, base0):
    pw = npts // NW
    nchunk = pw // CP
    mesh = plsc.VectorSubcoreMesh(core_axis_name="c", subcore_axis_name="s")

    @functools.partial(
        pl.kernel,
        mesh=mesh,
        out_type=jax.ShapeDtypeStruct((npts, C1 + C2), jnp.float32),
        scratch_types=[
            pltpu.VMEM((CP,), jnp.int32),
            pltpu.VMEM((CP,), jnp.int32),
            pltpu.VMEM((CP,), jnp.int32),
            pltpu.VMEM((CP,), jnp.float32),
            pltpu.VMEM((CP,), jnp.float32),
            pltpu.VMEM((CP,), jnp.float32),
            pltpu.VMEM((CP, C2), jnp.float32),
            pltpu.VMEM((CP, C2), jnp.float32),
            pltpu.VMEM((CP, C2), jnp.float32),
            pltpu.VMEM((CP, C2), jnp.float32),
            pltpu.VMEM((CP, C1), jnp.float32),
            pltpu.SemaphoreType.DMA,
        ],
    )
    def _sc_interp(table_hbm, i0_hbm, i1_hbm, i2_hbm, w0_hbm, w1_hbm, w2_hbm,
                   uf_hbm, out_hbm,
                   i0_v, i1_v, i2_v, w0_v, w1_v, w2_v,
                   g0_v, g1_v, g2_v, acc_v, uf_v, sem):
        wid = lax.axis_index("s") * 2 + lax.axis_index("c")
        wbase = wid * pw

        def chunk_body(t, carry):
            base = wbase + t * CP
            gbase = base0 + base
            pltpu.sync_copy(i0_hbm.at[pl.ds(base, CP)], i0_v)
            pltpu.sync_copy(i1_hbm.at[pl.ds(base, CP)], i1_v)
            pltpu.sync_copy(i2_hbm.at[pl.ds(base, CP)], i2_v)
            pltpu.sync_copy(w0_hbm.at[pl.ds(base, CP)], w0_v)
            pltpu.sync_copy(w1_hbm.at[pl.ds(base, CP)], w1_v)
            pltpu.sync_copy(w2_hbm.at[pl.ds(base, CP)], w2_v)
            pltpu.sync_copy(uf_hbm.at[pl.ds(gbase, CP), :], uf_v)
            a0 = pltpu.async_copy(table_hbm.at[i0_v], g0_v, sem)
            a1 = pltpu.async_copy(table_hbm.at[i1_v], g1_v, sem)
            a2 = pltpu.async_copy(table_hbm.at[i2_v], g2_v, sem)
            a0.wait()
            a1.wait()
            a2.wait()

            def group_body(q, carry2):
                wq0 = w0_v[pl.ds(q * 16, 16)]
                wq1 = w1_v[pl.ds(q * 16, 16)]
                wq2 = w2_v[pl.ds(q * 16, 16)]
                for j in range(16):
                    p = q * 16 + j
                    w0 = _bcast(wq0, j)
                    w1 = _bcast(wq1, j)
                    w2 = _bcast(wq2, j)
                    for f in range(C2 // 16):
                        sl = pl.ds(f * 16, 16)
                        acc_v[p, sl] = (w0 * g0_v[p, sl] + w1 * g1_v[p, sl]
                                        + w2 * g2_v[p, sl])
                return carry2

            lax.fori_loop(0, CP // 16, group_body, 0)
            pltpu.sync_copy(acc_v, out_hbm.at[pl.ds(base, CP), pl.ds(0, C2)])
            pltpu.sync_copy(uf_v, out_hbm.at[pl.ds(base, CP), pl.ds(C2, C1)])
            return carry

        lax.fori_loop(0, nchunk, chunk_body, 0)

    return _sc_interp


NSPLIT = 2
HB = B // NSPLIT
HP = HB * N


@jax.jit
def kernel(unknown, known, unknow_feats, known_feats):
    table = known_feats.reshape(B * M, C2)
    uf_flat = unknow_feats.reshape(BN, C1)
    parts = []
    for h in range(NSPLIT):
        bs = slice(h * HB, (h + 1) * HB)
        i0, i1, i2, w0, w1, w2 = _three_nn(unknown[bs], known[bs], h * HB, HB)
        parts.append(_build_sc_interp(HP, h * HP)(
            table, i0, i1, i2, w0, w1, w2, uf_flat))
    out = jnp.concatenate(parts, axis=0).reshape(B, N, C1 + C2)
    return (out, out)
